# Initial kernel scaffold; baseline (speedup 1.0000x reference)
import jax
import jax.numpy as jnp
from jax.experimental import pallas as pl


def kernel(obs, w1, b1, w2, b2, wself, wmsg, bl, wout, bout):
    raise NotImplementedError("write your pallas kernel here")


# speedup vs baseline: 12.7890x; 12.7890x over previous
import numpy as np
import jax
import jax.numpy as jnp
from jax.experimental import pallas as pl
from jax.experimental.pallas import tpu as pltpu

B = 16384
H = 128
L = 8
NJ = 17
N = 13
OBS = 348
TB = 512


def _node_idx():
    base = {0: list(range(0, 5)) + list(range(22, 28)),
            1: list(range(5, 7)) + list(range(28, 30)),
            2: [7, 30],
            3: list(range(8, 12)) + list(range(31, 35)),
            4: [11, 34],
            5: [],
            6: list(range(12, 16)) + list(range(35, 39)),
            7: [15, 38],
            8: [],
            9: list(range(16, 19)) + list(range(39, 42)),
            10: [18, 41],
            11: list(range(19, 22)) + list(range(42, 45)),
            12: [21, 44]}
    qfrc = {2: [0, 1, 2], 3: [3, 4, 5], 4: [6], 6: [7, 8, 9], 7: [10],
            9: [11, 12], 10: [13], 11: [14, 15], 12: [16]}
    out = []
    for k in range(13):
        idx = list(base[k])
        idx += list(range(45 + k * 10, 45 + (k + 1) * 10))
        idx += list(range(175 + k * 6, 175 + (k + 1) * 6))
        if k in qfrc:
            idx += [253 + i for i in qfrc[k]]
        idx += list(range(270 + k * 6, 270 + (k + 1) * 6))
        out.append(np.array(idx, dtype=np.int32))
    return out


_IDX = _node_idx()

_NBR = [[1, 9, 11], [0, 2], [1, 3, 6], [2, 4], [3, 5], [4], [2, 7],
        [6, 8], [7], [0, 10], [9], [0, 12], [11]]

_JMAP = np.array([[1, 2], [1, 2], [1, 2], [2, 3], [2, 3], [2, 3], [3, 4],
                  [2, 6], [2, 6], [2, 6], [6, 7], [0, 9], [0, 9], [9, 10],
                  [0, 11], [0, 11], [11, 12]])

_SCAT = []
for _k in range(N):
    _s = np.zeros((OBS, _IDX[_k].shape[0]), dtype=np.float32)
    _s[_IDX[_k], np.arange(_IDX[_k].shape[0])] = 1.0
    _SCAT.append(_s)

_M0 = np.zeros((NJ, N), dtype=np.float32)
_M1 = np.zeros((NJ, N), dtype=np.float32)
_M0[np.arange(NJ), _JMAP[:, 0]] = 1.0
_M1[np.arange(NJ), _JMAP[:, 1]] = 1.0


def _elu(x):
    return jnp.where(x > 0, x, jnp.exp(x) - 1.0)


def _gnn_kernel(obs_ref, w1_ref, b1_ref, w2_ref, b2_ref, ws_ref, wm_ref,
                bl_ref, wr_ref, bout_ref, out_ref, h_scr, msg_scr):
    h1 = jnp.dot(obs_ref[:], w1_ref[:],
                 preferred_element_type=jnp.float32) + b1_ref[:]
    h1 = _elu(h1)
    for k in range(N):
        blk = h1[:, k * H:(k + 1) * H]
        h_scr[k * TB:(k + 1) * TB, :] = (
            jnp.dot(blk, w2_ref[k], preferred_element_type=jnp.float32)
            + b2_ref[k])
    for l in range(L):
        for n in range(N):
            acc = h_scr[_NBR[n][0] * TB:(_NBR[n][0] + 1) * TB, :]
            for m in _NBR[n][1:]:
                acc = acc + h_scr[m * TB:(m + 1) * TB, :]
            msg_scr[n * TB:(n + 1) * TB, :] = acc
        a = jnp.dot(h_scr[:], ws_ref[l], preferred_element_type=jnp.float32)
        m = jnp.dot(msg_scr[:], wm_ref[l], preferred_element_type=jnp.float32)
        h_scr[:] = _elu(a + m + bl_ref[l])
    out = jnp.zeros((TB, NJ), dtype=jnp.float32) + bout_ref[:]
    for n in range(N):
        out = out + jnp.dot(h_scr[n * TB:(n + 1) * TB, :], wr_ref[n],
                            preferred_element_type=jnp.float32)
    out_ref[:] = out


def kernel(obs, w1, b1, w2, b2, wself, wmsg, bl, wout, bout):
    w1cat = jnp.concatenate(
        [jnp.asarray(_SCAT[k]) @ w1[k] for k in range(N)], axis=1)
    b1cat = b1.reshape(1, N * H)
    wr = (jnp.einsum('jn,jh->nhj', jnp.asarray(_M0), wout[:, :H]) +
          jnp.einsum('jn,jh->nhj', jnp.asarray(_M1), wout[:, H:]))
    boutr = bout.reshape(1, NJ)

    grid = (B // TB,)
    out = pl.pallas_call(
        _gnn_kernel,
        grid=grid,
        in_specs=[
            pl.BlockSpec((TB, OBS), lambda i: (i, 0)),
            pl.BlockSpec((OBS, N * H), lambda i: (0, 0)),
            pl.BlockSpec((1, N * H), lambda i: (0, 0)),
            pl.BlockSpec((N, H, H), lambda i: (0, 0, 0)),
            pl.BlockSpec((N, H), lambda i: (0, 0)),
            pl.BlockSpec((L, H, H), lambda i: (0, 0, 0)),
            pl.BlockSpec((L, H, H), lambda i: (0, 0, 0)),
            pl.BlockSpec((L, H), lambda i: (0, 0)),
            pl.BlockSpec((N, H, NJ), lambda i: (0, 0, 0)),
            pl.BlockSpec((1, NJ), lambda i: (0, 0)),
        ],
        out_specs=pl.BlockSpec((TB, NJ), lambda i: (i, 0)),
        out_shape=jax.ShapeDtypeStruct((B, NJ), jnp.float32),
        scratch_shapes=[
            pltpu.VMEM((N * TB, H), jnp.float32),
            pltpu.VMEM((N * TB, H), jnp.float32),
        ],
    )(obs, w1cat, b1cat, w2, b2, wself, wmsg, bl, wr, boutr)
    return out
